# baseline (speedup 1.0000x reference)
import functools

import jax
import jax.numpy as jnp
from jax import lax
from jax.experimental import pallas as pl
from jax.experimental.pallas import tpu as pltpu
from jax.experimental.pallas import tpu_sc as plsc

_NCLS = 19
_B, _H, _W = 8, 512, 512
_N = _B * _H * _W
_ROWS = 512
_FLAG = 32
_NBINS = 64
_LANES = 16

_SC_CORES = 2
_SC_SUBCORES = 16
_NTILES = _SC_CORES * _SC_SUBCORES
_NROWS = _N // _W
_TILE_ROWS = (_NROWS // 2) // _NTILES
_CHUNK_ROWS = 32
_NCHUNKS = _TILE_ROWS // _CHUNK_ROWS


def _dense_body(x_ref, t_ref, o_ref, c_ref):
    x = x_ref[0]
    t = t_ref[0, 0]
    m = jnp.max(x, axis=0)
    ex = jnp.exp(x - m[None])
    lse = jnp.log(jnp.sum(ex, axis=0)) + m
    cls = lax.broadcasted_iota(jnp.int32, x.shape, 0)
    onehot = cls == t[None]
    x_t = jnp.sum(jnp.where(onehot, x, 0.0), axis=0)
    first_arg = jnp.min(jnp.where(x == m[None], cls, _NCLS), axis=0)
    o_ref[...] = lse - x_t
    lane = lax.broadcasted_iota(jnp.int32, (_ROWS, _W), 1) & (_LANES - 1)
    mis = (first_arg != t).astype(jnp.int32)
    c_ref[...] = t + _FLAG * mis + _NBINS * lane


def _tc_ce_comb(inp, tgt, half, interpret=False):
    bh = _B // 2
    grid = (bh, _H // _ROWS)
    return pl.pallas_call(
        _dense_body,
        grid=grid,
        in_specs=[
            pl.BlockSpec((1, _NCLS, _ROWS, _W),
                         lambda b, r: (half * bh + b, 0, r, 0)),
            pl.BlockSpec((1, 1, _ROWS, _W),
                         lambda b, r: (half * bh + b, 0, r, 0)),
        ],
        out_specs=[
            pl.BlockSpec((_ROWS, _W), lambda b, r: (b * (_H // _ROWS) + r, 0)),
            pl.BlockSpec((_ROWS, _W), lambda b, r: (b * (_H // _ROWS) + r, 0)),
        ],
        out_shape=[
            jax.ShapeDtypeStruct((_NROWS // 2, _W), jnp.float32),
            jax.ShapeDtypeStruct((_NROWS // 2, _W), jnp.int32),
        ],
        compiler_params=pltpu.CompilerParams(
            dimension_semantics=("parallel", "parallel")),
        interpret=interpret,
    )(inp, tgt)


_UNROLL = 8


def _sc_body(vals_hbm, comb_hbm, out_hbm, vbuf0, vbuf1, cbuf0, cbuf1,
             hce_v, hcnt_v, pub, sem0, sem1):
    cid = lax.axis_index("c")
    sid = lax.axis_index("s")
    wid = sid * _SC_CORES + cid
    zf = jnp.zeros((_LANES,), jnp.float32)
    for j in range(_LANES * _NBINS // _LANES):
        hce_v[pl.ds(j * _LANES, _LANES)] = zf
        hcnt_v[pl.ds(j * _LANES, _LANES)] = zf

    base = wid * _TILE_ROWS
    ones_f = jnp.full((_LANES,), 1.0, jnp.float32)

    vbufs, cbufs, sems = (vbuf0, vbuf1), (cbuf0, cbuf1), (sem0, sem1)

    def start(k):
        b = k % 2
        cv = pltpu.async_copy(
            vals_hbm.at[pl.ds(base + k * _CHUNK_ROWS, _CHUNK_ROWS)],
            vbufs[b], sems[b])
        ct = pltpu.async_copy(
            comb_hbm.at[pl.ds(base + k * _CHUNK_ROWS, _CHUNK_ROWS)],
            cbufs[b], sems[b])
        return cv, ct

    pending = start(0)
    for k in range(_NCHUNKS):
        nxt = start(k + 1) if k + 1 < _NCHUNKS else None
        pending[0].wait()
        pending[1].wait()
        vbuf, cbuf = vbufs[k % 2], cbufs[k % 2]

        def step(r, c):
            for g in range(_W // (_LANES * _UNROLL)):
                loaded = []
                for u in range(_UNROLL):
                    col = g * (_LANES * _UNROLL) + u * _LANES
                    loaded.append((vbuf[r, pl.ds(col, _LANES)],
                                   cbuf[r, pl.ds(col, _LANES)]))
                for vv, cv in loaded:
                    plsc.addupdate_scatter(hce_v, [cv], vv)
                    plsc.addupdate_scatter(hcnt_v, [cv], ones_f)
            return c

        lax.fori_loop(0, _CHUNK_ROWS, step, 0)
        pending = nxt

    for j in range(_NBINS // _LANES):
        acc_ce = zf
        acc_cnt = zf
        for l in range(_LANES):
            acc_ce = acc_ce + hce_v[pl.ds(l * _NBINS + j * _LANES, _LANES)]
            acc_cnt = acc_cnt + hcnt_v[pl.ds(l * _NBINS + j * _LANES, _LANES)]
        pub[pl.ds(j * _LANES, _LANES)] = acc_ce
        pub[pl.ds(_NBINS + j * _LANES, _LANES)] = acc_cnt
    pltpu.sync_copy(pub, out_hbm.at[wid])


def _sc_finish_body(part0_hbm, part1_hbm, out_hbm, pbuf, obuf):
    cid = lax.axis_index("c")
    sid = lax.axis_index("s")

    @pl.when((cid == 0) & (sid == 0))
    def _():
        pltpu.sync_copy(part0_hbm, pbuf.at[pl.ds(0, _NTILES)])
        pltpu.sync_copy(part1_hbm, pbuf.at[pl.ds(_NTILES, _NTILES)])
        zf = jnp.zeros((_LANES,), jnp.float32)
        one = jnp.full((_LANES,), 1.0, jnp.float32)
        ngrp = _NBINS // _LANES
        h = [[zf for _ in range(ngrp)] for _ in range(2)]
        for tt in range(2 * _NTILES):
            for r in range(2):
                for j in range(ngrp):
                    h[r][j] = h[r][j] + pbuf[tt, pl.ds(r * _NBINS + j * _LANES, _LANES)]
        hce, hcnt = h
        total = zf
        for j in range(2):
            s_j = hce[j] + hce[j + 2]
            gt = hcnt[j] + hcnt[j + 2]
            fn = hcnt[j + 2]
            w = jnp.where(fn > 0, fn, one) / jnp.where(gt > 0, gt, one)
            total = total + w * s_j
        loss = jnp.sum(total) * (1.0 / _N)
        obuf[...] = jnp.broadcast_to(loss, (_LANES,))
        pltpu.sync_copy(obuf, out_hbm)


def _sc_mesh():
    return plsc.VectorSubcoreMesh(core_axis_name="c", subcore_axis_name="s",
                                  num_cores=_SC_CORES,
                                  num_subcores=_SC_SUBCORES)


@functools.cache
def _sc_hist():
    return pl.kernel(
        _sc_body,
        out_type=jax.ShapeDtypeStruct((_NTILES, 2 * _NBINS), jnp.float32),
        mesh=_sc_mesh(),
        compiler_params=pltpu.CompilerParams(needs_layout_passes=False),
        scratch_types=[
            pltpu.VMEM((_CHUNK_ROWS, _W), jnp.float32),
            pltpu.VMEM((_CHUNK_ROWS, _W), jnp.float32),
            pltpu.VMEM((_CHUNK_ROWS, _W), jnp.int32),
            pltpu.VMEM((_CHUNK_ROWS, _W), jnp.int32),
            pltpu.VMEM((_LANES * _NBINS,), jnp.float32),
            pltpu.VMEM((_LANES * _NBINS,), jnp.float32),
            pltpu.VMEM((2 * _NBINS,), jnp.float32),
            pltpu.SemaphoreType.DMA,
            pltpu.SemaphoreType.DMA,
        ],
    )


@functools.cache
def _sc_finish():
    return pl.kernel(
        _sc_finish_body,
        out_type=jax.ShapeDtypeStruct((_LANES,), jnp.float32),
        mesh=_sc_mesh(),
        compiler_params=pltpu.CompilerParams(needs_layout_passes=False),
        scratch_types=[
            pltpu.VMEM((2 * _NTILES, 2 * _NBINS), jnp.float32),
            pltpu.VMEM((_LANES,), jnp.float32),
        ],
    )


def kernel(input, target):
    ce0, comb0 = _tc_ce_comb(input, target, 0)
    parts0 = _sc_hist()(ce0, comb0)
    ce1, comb1 = _tc_ce_comb(input, target, 1)
    parts1 = _sc_hist()(ce1, comb1)
    return _sc_finish()(parts0, parts1)[0]


# speedup vs baseline: 1.0628x; 1.0628x over previous
import functools

import jax
import jax.numpy as jnp
from jax import lax
from jax.experimental import pallas as pl
from jax.experimental.pallas import tpu as pltpu
from jax.experimental.pallas import tpu_sc as plsc

_NCLS = 19
_B, _H, _W = 8, 512, 512
_N = _B * _H * _W
_ROWS = 256
_FLAG = 32
_NBINS = 64
_LANES = 16

_SC_CORES = 2
_SC_SUBCORES = 16
_NTILES = _SC_CORES * _SC_SUBCORES
_NROWS = _N // _W
_B0 = 6
_B1 = _B - _B0


def _dense_body(x_ref, t_ref, o_ref, c_ref):
    x = x_ref[0]
    t = t_ref[0, 0]
    m = jnp.max(x, axis=0)
    ex = jnp.exp(x - m[None])
    lse = jnp.log(jnp.sum(ex, axis=0)) + m
    cls = lax.broadcasted_iota(jnp.int32, x.shape, 0)
    onehot = cls == t[None]
    x_t = jnp.sum(jnp.where(onehot, x, 0.0), axis=0)
    first_arg = jnp.min(jnp.where(x == m[None], cls, _NCLS), axis=0)
    o_ref[...] = lse - x_t
    lane = lax.broadcasted_iota(jnp.int32, (_ROWS, _W), 1) & (_LANES - 1)
    mis = (first_arg != t).astype(jnp.int32)
    c_ref[...] = t + _FLAG * mis + _NBINS * lane


def _tc_ce_comb(inp, tgt, b0, nb, interpret=False):
    grid = (nb, _H // _ROWS)
    return pl.pallas_call(
        _dense_body,
        grid=grid,
        in_specs=[
            pl.BlockSpec((1, _NCLS, _ROWS, _W),
                         lambda b, r: (b0 + b, 0, r, 0)),
            pl.BlockSpec((1, 1, _ROWS, _W),
                         lambda b, r: (b0 + b, 0, r, 0)),
        ],
        out_specs=[
            pl.BlockSpec((_ROWS, _W), lambda b, r: (b * (_H // _ROWS) + r, 0)),
            pl.BlockSpec((_ROWS, _W), lambda b, r: (b * (_H // _ROWS) + r, 0)),
        ],
        out_shape=[
            jax.ShapeDtypeStruct((nb * _H, _W), jnp.float32),
            jax.ShapeDtypeStruct((nb * _H, _W), jnp.int32),
        ],
        compiler_params=pltpu.CompilerParams(
            dimension_semantics=("parallel", "parallel")),
        interpret=interpret,
    )(inp, tgt)


_UNROLL = 8


def _make_sc_body(tile_rows, chunk_rows):
    nchunks = tile_rows // chunk_rows

    def _sc_body(vals_hbm, comb_hbm, out_hbm, vbuf0, vbuf1, cbuf0, cbuf1,
                 hce_v, hcnt_v, pub, sem0, sem1):
        cid = lax.axis_index("c")
        sid = lax.axis_index("s")
        wid = sid * _SC_CORES + cid
        zf = jnp.zeros((_LANES,), jnp.float32)
        for j in range(_LANES * _NBINS // _LANES):
            hce_v[pl.ds(j * _LANES, _LANES)] = zf
            hcnt_v[pl.ds(j * _LANES, _LANES)] = zf

        base = wid * tile_rows
        ones_f = jnp.full((_LANES,), 1.0, jnp.float32)

        vbufs, cbufs, sems = (vbuf0, vbuf1), (cbuf0, cbuf1), (sem0, sem1)

        def start(k):
            b = k % 2
            cv = pltpu.async_copy(
                vals_hbm.at[pl.ds(base + k * chunk_rows, chunk_rows)],
                vbufs[b], sems[b])
            ct = pltpu.async_copy(
                comb_hbm.at[pl.ds(base + k * chunk_rows, chunk_rows)],
                cbufs[b], sems[b])
            return cv, ct

        pending = start(0)
        for k in range(nchunks):
            nxt = start(k + 1) if k + 1 < nchunks else None
            pending[0].wait()
            pending[1].wait()
            vbuf, cbuf = vbufs[k % 2], cbufs[k % 2]

            def step(r, c):
                for g in range(_W // (_LANES * _UNROLL)):
                    loaded = []
                    for u in range(_UNROLL):
                        col = g * (_LANES * _UNROLL) + u * _LANES
                        loaded.append((vbuf[r, pl.ds(col, _LANES)],
                                       cbuf[r, pl.ds(col, _LANES)]))
                    for vv, cv in loaded:
                        plsc.addupdate_scatter(hce_v, [cv], vv)
                        plsc.addupdate_scatter(hcnt_v, [cv], ones_f)
                return c

            lax.fori_loop(0, chunk_rows, step, 0)
            pending = nxt

        for j in range(_NBINS // _LANES):
            acc_ce = zf
            acc_cnt = zf
            for l in range(_LANES):
                acc_ce = acc_ce + hce_v[pl.ds(l * _NBINS + j * _LANES, _LANES)]
                acc_cnt = acc_cnt + hcnt_v[pl.ds(l * _NBINS + j * _LANES, _LANES)]
            pub[pl.ds(j * _LANES, _LANES)] = acc_ce
            pub[pl.ds(_NBINS + j * _LANES, _LANES)] = acc_cnt
        pltpu.sync_copy(pub, out_hbm.at[wid])

    return _sc_body


def _sc_finish_body(part0_hbm, part1_hbm, out_hbm, pbuf, obuf):
    cid = lax.axis_index("c")
    sid = lax.axis_index("s")

    @pl.when((cid == 0) & (sid == 0))
    def _():
        pltpu.sync_copy(part0_hbm, pbuf.at[pl.ds(0, _NTILES)])
        pltpu.sync_copy(part1_hbm, pbuf.at[pl.ds(_NTILES, _NTILES)])
        zf = jnp.zeros((_LANES,), jnp.float32)
        one = jnp.full((_LANES,), 1.0, jnp.float32)
        ngrp = _NBINS // _LANES
        h = [[zf for _ in range(ngrp)] for _ in range(2)]
        for tt in range(2 * _NTILES):
            for r in range(2):
                for j in range(ngrp):
                    h[r][j] = h[r][j] + pbuf[tt, pl.ds(r * _NBINS + j * _LANES, _LANES)]
        hce, hcnt = h
        total = zf
        for j in range(2):
            s_j = hce[j] + hce[j + 2]
            gt = hcnt[j] + hcnt[j + 2]
            fn = hcnt[j + 2]
            w = jnp.where(fn > 0, fn, one) / jnp.where(gt > 0, gt, one)
            total = total + w * s_j
        loss = jnp.sum(total) * (1.0 / _N)
        obuf[...] = jnp.broadcast_to(loss, (_LANES,))
        pltpu.sync_copy(obuf, out_hbm)


def _sc_mesh():
    return plsc.VectorSubcoreMesh(core_axis_name="c", subcore_axis_name="s",
                                  num_cores=_SC_CORES,
                                  num_subcores=_SC_SUBCORES)


@functools.cache
def _sc_hist(tile_rows, chunk_rows):
    return pl.kernel(
        _make_sc_body(tile_rows, chunk_rows),
        out_type=jax.ShapeDtypeStruct((_NTILES, 2 * _NBINS), jnp.float32),
        mesh=_sc_mesh(),
        compiler_params=pltpu.CompilerParams(needs_layout_passes=False),
        scratch_types=[
            pltpu.VMEM((chunk_rows, _W), jnp.float32),
            pltpu.VMEM((chunk_rows, _W), jnp.float32),
            pltpu.VMEM((chunk_rows, _W), jnp.int32),
            pltpu.VMEM((chunk_rows, _W), jnp.int32),
            pltpu.VMEM((_LANES * _NBINS,), jnp.float32),
            pltpu.VMEM((_LANES * _NBINS,), jnp.float32),
            pltpu.VMEM((2 * _NBINS,), jnp.float32),
            pltpu.SemaphoreType.DMA,
            pltpu.SemaphoreType.DMA,
        ],
    )


@functools.cache
def _sc_finish():
    return pl.kernel(
        _sc_finish_body,
        out_type=jax.ShapeDtypeStruct((_LANES,), jnp.float32),
        mesh=_sc_mesh(),
        compiler_params=pltpu.CompilerParams(needs_layout_passes=False),
        scratch_types=[
            pltpu.VMEM((2 * _NTILES, 2 * _NBINS), jnp.float32),
            pltpu.VMEM((_LANES,), jnp.float32),
        ],
    )


def kernel(input, target):
    ce0, comb0 = _tc_ce_comb(input, target, 0, _B0)
    parts0 = _sc_hist(_B0 * _H // _NTILES, 32)(ce0, comb0)
    ce1, comb1 = _tc_ce_comb(input, target, _B0, _B1)
    parts1 = _sc_hist(_B1 * _H // _NTILES, 16)(ce1, comb1)
    return _sc_finish()(parts0, parts1)[0]
